# baseline (speedup 1.0000x reference)
import functools

import jax
import jax.numpy as jnp
from jax import lax
from jax.experimental import pallas as pl
from jax.experimental.pallas import tpu as pltpu
from jax.experimental.pallas import tpu_sc as plsc

N_FIELDS = 26
VOCAB = 100000
EMB = 16
OHE = 100
HID = 128
BATCH = 16384
EMB_FEAT = N_FIELDS * EMB
TOTAL_ROWS = BATCH * N_FIELDS

_NC = 2
_NS = 16
_NW = _NC * _NS

_C = 1408
_CHUNKS = 99968 // _C
_TAIL0 = _CHUNKS * _C
_TAILW = VOCAB - _TAIL0
_N_U1 = N_FIELDS * _CHUNKS
_U1_PER_W = (_N_U1 + _NW - 1) // _NW


_U1_GROUP = 6
_U1_ROUNDS = (_U1_PER_W + _U1_GROUP - 1) // _U1_GROUP


def _relayout_body(
    tab_hbm, out_hbm, in0, in1, in2, ot0, ot1, in_t, out_t, si0, si1, si2, so0, so1
):
    wid = lax.axis_index("s") * _NC + lax.axis_index("c")
    rows_iota = lax.broadcasted_iota(jnp.int32, (EMB,), 0)
    ins = ((in0, si0), (in1, si1), (in2, si2))
    outs = ((ot0, so0), (ot1, so1))

    def in_slice(u):
        f = u // _CHUNKS
        c0 = (u % _CHUNKS) * _C
        return tab_hbm.at[pl.ds(f * EMB, EMB), pl.ds(c0, _C)]

    def out_slice(u):
        f = u // _CHUNKS
        c0 = (u % _CHUNKS) * _C
        return out_hbm.at[pl.ds((f * VOCAB + c0) * EMB, _C * EMB)]

    def transpose(src, dst, width):
        @plsc.parallel_loop(0, width // EMB, unroll=4, carry=rows_iota * EMB)
        def body(cb, idx0):
            idx = idx0
            for j in range(EMB):
                vec = src[j, pl.ds(cb * EMB, EMB)]
                plsc.store_scatter(dst, [idx], vec)
                idx = idx + 1
            return idx0 + EMB * EMB

    for b in range(3):
        u = wid + _NW * b

        @pl.when(u < _N_U1)
        def _():
            pltpu.async_copy(in_slice(u), ins[b][0], ins[b][1])

    def round_body(m, carry):
        for b in range(_U1_GROUP):
            k = _U1_GROUP * m + b
            u = wid + _NW * k
            u_prev = u - 2 * _NW
            u_next = u + 3 * _NW
            ib, ob = ins[b % 3], outs[b % 2]

            @pl.when((k >= 2) & (u_prev < _N_U1))
            def _():
                pltpu.make_async_copy(ob[0], out_slice(u_prev), ob[1]).wait()

            @pl.when(u < _N_U1)
            def _():
                pltpu.make_async_copy(in_slice(u), ib[0], ib[1]).wait()
                transpose(ib[0], ob[0], _C)
                pltpu.async_copy(ob[0], out_slice(u), ob[1])

            @pl.when(u_next < _N_U1)
            def _():
                pltpu.async_copy(in_slice(u_next), ib[0], ib[1])

        return carry

    lax.fori_loop(0, _U1_ROUNDS, round_body, 0)

    @pl.when(wid < N_FIELDS)
    def _():
        f = wid
        pltpu.sync_copy(tab_hbm.at[pl.ds(f * EMB, EMB), pl.ds(_TAIL0, _TAILW)], in_t)
        transpose(in_t, out_t, _TAILW)
        pltpu.sync_copy(out_t, out_hbm.at[pl.ds((f * VOCAB + _TAIL0) * EMB, _TAILW * EMB)])


_relayout = functools.partial(
    pl.kernel,
    mesh=plsc.VectorSubcoreMesh(core_axis_name="c", subcore_axis_name="s"),
    out_type=jax.ShapeDtypeStruct((N_FIELDS * VOCAB * EMB,), jnp.float32),
    scratch_types=[
        pltpu.VMEM((EMB, _C), jnp.float32),
        pltpu.VMEM((EMB, _C), jnp.float32),
        pltpu.VMEM((EMB, _C), jnp.float32),
        pltpu.VMEM((_C * EMB,), jnp.float32),
        pltpu.VMEM((_C * EMB,), jnp.float32),
        pltpu.VMEM((EMB, _TAILW), jnp.float32),
        pltpu.VMEM((_TAILW * EMB,), jnp.float32),
        pltpu.SemaphoreType.DMA,
        pltpu.SemaphoreType.DMA,
        pltpu.SemaphoreType.DMA,
        pltpu.SemaphoreType.DMA,
        pltpu.SemaphoreType.DMA,
    ],
    compiler_params=pltpu.CompilerParams(
        use_tc_tiling_on_sc=True, needs_layout_passes=False
    ),
)(_relayout_body)


_PER_W = TOTAL_ROWS // _NW
_CHUNK = 3328
_N_CHUNKS = _PER_W // _CHUNK


_G_CHUNK = 1664
_HALF_ROWS = TOTAL_ROWS // 2
_PER_W_H = _HALF_ROWS // _NW
_G_N = _PER_W_H // _G_CHUNK


def _gather_body(table_hbm, idx_hbm, out_hbm, i0, i1, r0, r1, gi0, gi1, gr0, gr1):
    wid = lax.axis_index("s") * _NC + lax.axis_index("c")
    base = wid * _PER_W_H
    idx = ((i0, gi0), (i1, gi1))
    rows = ((r0, gr0), (r1, gr1))

    def islice(c):
        return idx_hbm.at[pl.ds(base + c * _G_CHUNK, _G_CHUNK)]

    def oslice(c):
        return out_hbm.at[pl.ds(base + c * _G_CHUNK, _G_CHUNK)]

    for b in range(2):
        pltpu.async_copy(islice(b), idx[b][0], idx[b][1])
    for c in range(_G_N):
        b = c % 2
        if c >= 2:
            pltpu.make_async_copy(rows[b][0], oslice(c - 2), rows[b][1]).wait()
        pltpu.make_async_copy(islice(c), idx[b][0], idx[b][1]).wait()
        pltpu.async_copy(table_hbm.at[idx[b][0]], rows[b][0], rows[b][1]).wait()
        pltpu.async_copy(rows[b][0], oslice(c), rows[b][1])
        if c + 2 < _G_N:
            pltpu.async_copy(islice(c + 2), idx[b][0], idx[b][1])
    for c in (_G_N - 2, _G_N - 1):
        pltpu.make_async_copy(rows[c % 2][0], oslice(c), rows[c % 2][1]).wait()


_gather = functools.partial(
    pl.kernel,
    mesh=plsc.VectorSubcoreMesh(core_axis_name="c", subcore_axis_name="s"),
    out_type=jax.ShapeDtypeStruct((_HALF_ROWS, EMB), jnp.float32),
    scratch_types=[
        pltpu.VMEM((_G_CHUNK,), jnp.int32),
        pltpu.VMEM((_G_CHUNK,), jnp.int32),
        pltpu.VMEM((_G_CHUNK, EMB), jnp.float32),
        pltpu.VMEM((_G_CHUNK, EMB), jnp.float32),
        pltpu.SemaphoreType.DMA,
        pltpu.SemaphoreType.DMA,
        pltpu.SemaphoreType.DMA,
        pltpu.SemaphoreType.DMA,
    ],
    compiler_params=pltpu.CompilerParams(use_tc_tiling_on_sc=False),
)(_gather_body)


_BM = 2048


def _mm0_body(o_ref, w2_ref, b_ref, out_ref):
    out_ref[...] = (
        jnp.dot(o_ref[...], w2_ref[...], preferred_element_type=jnp.float32)
        + b_ref[...]
    )


_mm0 = pl.pallas_call(
    _mm0_body,
    grid=(BATCH // _BM,),
    in_specs=[
        pl.BlockSpec((_BM, OHE), lambda i: (i, 0)),
        pl.BlockSpec((OHE, HID), lambda i: (0, 0)),
        pl.BlockSpec((1, HID), lambda i: (0, 0)),
    ],
    out_specs=pl.BlockSpec((_BM, HID), lambda i: (i, 0)),
    out_shape=jax.ShapeDtypeStruct((BATCH, HID), jnp.float32),
)


def _mm_body(e_ref, h0_ref, w1_ref, out_ref):
    acc = jnp.dot(e_ref[...], w1_ref[...], preferred_element_type=jnp.float32)
    out_ref[...] = acc + h0_ref[...]


_mm = pl.pallas_call(
    _mm_body,
    grid=(BATCH // 2 // _BM,),
    in_specs=[
        pl.BlockSpec((_BM, EMB_FEAT), lambda i: (i, 0)),
        pl.BlockSpec((_BM, HID), lambda i: (i, 0)),
        pl.BlockSpec((EMB_FEAT, HID), lambda i: (0, 0)),
    ],
    out_specs=pl.BlockSpec((_BM, HID), lambda i: (i, 0)),
    out_shape=jax.ShapeDtypeStruct((BATCH // 2, HID), jnp.float32),
)


@jax.jit
def kernel(embed_idx, ohes, tables, W, b):
    tab_t = tables.transpose(0, 2, 1).reshape(EMB_FEAT, VOCAB)
    table_lin = _relayout(tab_t).reshape(N_FIELDS * VOCAB, EMB)
    offs = jnp.arange(N_FIELDS, dtype=jnp.int32) * VOCAB
    flat_idx = (embed_idx.astype(jnp.int32) + offs[None, :]).reshape(TOTAL_ROWS)
    h0 = _mm0(ohes, W[EMB_FEAT:], b.reshape(1, HID))
    w1 = W[:EMB_FEAT]
    halves = []
    for h in range(2):
        idx_h = lax.slice(flat_idx, (h * _HALF_ROWS,), ((h + 1) * _HALF_ROWS,))
        e_h = _gather(table_lin, idx_h).reshape(BATCH // 2, EMB_FEAT)
        h0_h = lax.slice(h0, (h * (BATCH // 2), 0), ((h + 1) * (BATCH // 2), HID))
        halves.append(_mm(e_h, h0_h, w1))
    return jnp.concatenate(halves, axis=0)


# speedup vs baseline: 1.0562x; 1.0562x over previous
import functools

import jax
import jax.numpy as jnp
from jax import lax
from jax.experimental import pallas as pl
from jax.experimental.pallas import tpu as pltpu
from jax.experimental.pallas import tpu_sc as plsc

N_FIELDS = 26
VOCAB = 100000
EMB = 16
OHE = 100
HID = 128
BATCH = 16384
EMB_FEAT = N_FIELDS * EMB
TOTAL_ROWS = BATCH * N_FIELDS

_NC = 2
_NS = 16
_NW = _NC * _NS

_C = 1408
_CHUNKS = 99968 // _C
_TAIL0 = _CHUNKS * _C
_TAILW = VOCAB - _TAIL0
_N_U1 = N_FIELDS * _CHUNKS
_U1_PER_W = (_N_U1 + _NW - 1) // _NW


_U1_GROUP = 6
_U1_ROUNDS = (_U1_PER_W + _U1_GROUP - 1) // _U1_GROUP


def _relayout_body(
    tab_hbm, out_hbm, in0, in1, in2, ot0, ot1, in_t, out_t, si0, si1, si2, so0, so1
):
    wid = lax.axis_index("s") * _NC + lax.axis_index("c")
    rows_iota = lax.broadcasted_iota(jnp.int32, (EMB,), 0)
    ins = ((in0, si0), (in1, si1), (in2, si2))
    outs = ((ot0, so0), (ot1, so1))

    def in_slice(u):
        f = u // _CHUNKS
        c0 = (u % _CHUNKS) * _C
        return tab_hbm.at[pl.ds(f * EMB, EMB), pl.ds(c0, _C)]

    def out_slice(u):
        f = u // _CHUNKS
        c0 = (u % _CHUNKS) * _C
        return out_hbm.at[pl.ds((f * VOCAB + c0) * EMB, _C * EMB)]

    def transpose(src, dst, width):
        @plsc.parallel_loop(0, width // EMB, unroll=4, carry=rows_iota * EMB)
        def body(cb, idx0):
            idx = idx0
            for j in range(EMB):
                vec = src[j, pl.ds(cb * EMB, EMB)]
                plsc.store_scatter(dst, [idx], vec)
                idx = idx + 1
            return idx0 + EMB * EMB

    for b in range(3):
        u = wid + _NW * b

        @pl.when(u < _N_U1)
        def _():
            pltpu.async_copy(in_slice(u), ins[b][0], ins[b][1])

    def round_body(m, carry):
        for b in range(_U1_GROUP):
            k = _U1_GROUP * m + b
            u = wid + _NW * k
            u_prev = u - 2 * _NW
            u_next = u + 3 * _NW
            ib, ob = ins[b % 3], outs[b % 2]

            @pl.when((k >= 2) & (u_prev < _N_U1))
            def _():
                pltpu.make_async_copy(ob[0], out_slice(u_prev), ob[1]).wait()

            @pl.when(u < _N_U1)
            def _():
                pltpu.make_async_copy(in_slice(u), ib[0], ib[1]).wait()
                transpose(ib[0], ob[0], _C)
                pltpu.async_copy(ob[0], out_slice(u), ob[1])

            @pl.when(u_next < _N_U1)
            def _():
                pltpu.async_copy(in_slice(u_next), ib[0], ib[1])

        return carry

    lax.fori_loop(0, _U1_ROUNDS, round_body, 0)

    @pl.when(wid < N_FIELDS)
    def _():
        f = wid
        pltpu.sync_copy(tab_hbm.at[pl.ds(f * EMB, EMB), pl.ds(_TAIL0, _TAILW)], in_t)
        transpose(in_t, out_t, _TAILW)
        pltpu.sync_copy(out_t, out_hbm.at[pl.ds((f * VOCAB + _TAIL0) * EMB, _TAILW * EMB)])


_relayout = functools.partial(
    pl.kernel,
    mesh=plsc.VectorSubcoreMesh(core_axis_name="c", subcore_axis_name="s"),
    out_type=jax.ShapeDtypeStruct((N_FIELDS * VOCAB * EMB,), jnp.float32),
    scratch_types=[
        pltpu.VMEM((EMB, _C), jnp.float32),
        pltpu.VMEM((EMB, _C), jnp.float32),
        pltpu.VMEM((EMB, _C), jnp.float32),
        pltpu.VMEM((_C * EMB,), jnp.float32),
        pltpu.VMEM((_C * EMB,), jnp.float32),
        pltpu.VMEM((EMB, _TAILW), jnp.float32),
        pltpu.VMEM((_TAILW * EMB,), jnp.float32),
        pltpu.SemaphoreType.DMA,
        pltpu.SemaphoreType.DMA,
        pltpu.SemaphoreType.DMA,
        pltpu.SemaphoreType.DMA,
        pltpu.SemaphoreType.DMA,
    ],
    compiler_params=pltpu.CompilerParams(
        use_tc_tiling_on_sc=True, needs_layout_passes=False
    ),
)(_relayout_body)


_PER_W = TOTAL_ROWS // _NW
_CHUNK = 3328
_N_CHUNKS = _PER_W // _CHUNK


_G_CHUNK = 1664
_G_N = _PER_W // _G_CHUNK


def _gather_body(table_hbm, idx_hbm, out_hbm, i0, i1, r0, r1, gi0, gi1, gr0, gr1):
    wid = lax.axis_index("s") * _NC + lax.axis_index("c")
    base = wid * _PER_W
    idx = ((i0, gi0), (i1, gi1))
    rows = ((r0, gr0), (r1, gr1))

    def islice(c):
        return idx_hbm.at[pl.ds(base + c * _G_CHUNK, _G_CHUNK)]

    def oslice(c):
        return out_hbm.at[pl.ds(base + c * _G_CHUNK, _G_CHUNK)]

    for b in range(2):
        pltpu.async_copy(islice(b), idx[b][0], idx[b][1])
    for c in range(_G_N):
        b = c % 2
        if c >= 2:
            pltpu.make_async_copy(rows[b][0], oslice(c - 2), rows[b][1]).wait()
        pltpu.make_async_copy(islice(c), idx[b][0], idx[b][1]).wait()
        pltpu.async_copy(table_hbm.at[idx[b][0]], rows[b][0], rows[b][1]).wait()
        pltpu.async_copy(rows[b][0], oslice(c), rows[b][1])
        if c + 2 < _G_N:
            pltpu.async_copy(islice(c + 2), idx[b][0], idx[b][1])
    for c in (_G_N - 2, _G_N - 1):
        pltpu.make_async_copy(rows[c % 2][0], oslice(c), rows[c % 2][1]).wait()


_gather = functools.partial(
    pl.kernel,
    mesh=plsc.VectorSubcoreMesh(core_axis_name="c", subcore_axis_name="s"),
    out_type=jax.ShapeDtypeStruct((TOTAL_ROWS, EMB), jnp.float32),
    scratch_types=[
        pltpu.VMEM((_G_CHUNK,), jnp.int32),
        pltpu.VMEM((_G_CHUNK,), jnp.int32),
        pltpu.VMEM((_G_CHUNK, EMB), jnp.float32),
        pltpu.VMEM((_G_CHUNK, EMB), jnp.float32),
        pltpu.SemaphoreType.DMA,
        pltpu.SemaphoreType.DMA,
        pltpu.SemaphoreType.DMA,
        pltpu.SemaphoreType.DMA,
    ],
    compiler_params=pltpu.CompilerParams(use_tc_tiling_on_sc=False),
)(_gather_body)


_BM = 2048


def _mm0_body(o_ref, w2_ref, b_ref, out_ref):
    out_ref[...] = (
        jnp.dot(o_ref[...], w2_ref[...], preferred_element_type=jnp.float32)
        + b_ref[...]
    )


_mm0 = pl.pallas_call(
    _mm0_body,
    grid=(BATCH // _BM,),
    in_specs=[
        pl.BlockSpec((_BM, OHE), lambda i: (i, 0)),
        pl.BlockSpec((OHE, HID), lambda i: (0, 0)),
        pl.BlockSpec((1, HID), lambda i: (0, 0)),
    ],
    out_specs=pl.BlockSpec((_BM, HID), lambda i: (i, 0)),
    out_shape=jax.ShapeDtypeStruct((BATCH, HID), jnp.float32),
)


def _mm_body(e_ref, h0_ref, w1_ref, out_ref):
    acc = jnp.dot(e_ref[...], w1_ref[...], preferred_element_type=jnp.float32)
    out_ref[...] = acc + h0_ref[...]


_mm = pl.pallas_call(
    _mm_body,
    grid=(BATCH // _BM,),
    in_specs=[
        pl.BlockSpec((_BM, EMB_FEAT), lambda i: (i, 0)),
        pl.BlockSpec((_BM, HID), lambda i: (i, 0)),
        pl.BlockSpec((EMB_FEAT, HID), lambda i: (0, 0)),
    ],
    out_specs=pl.BlockSpec((_BM, HID), lambda i: (i, 0)),
    out_shape=jax.ShapeDtypeStruct((BATCH, HID), jnp.float32),
)


@jax.jit
def kernel(embed_idx, ohes, tables, W, b):
    tab_t = tables.transpose(0, 2, 1).reshape(EMB_FEAT, VOCAB)
    table_lin = _relayout(tab_t).reshape(N_FIELDS * VOCAB, EMB)
    offs = jnp.arange(N_FIELDS, dtype=jnp.int32) * VOCAB
    flat_idx = (embed_idx.astype(jnp.int32) + offs[None, :]).reshape(TOTAL_ROWS)
    h0 = _mm0(ohes, W[EMB_FEAT:], b.reshape(1, HID))
    e = _gather(table_lin, flat_idx)
    e = e.reshape(BATCH, EMB_FEAT)
    return _mm(e, h0, W[:EMB_FEAT])


# speedup vs baseline: 1.0645x; 1.0079x over previous
import functools

import jax
import jax.numpy as jnp
from jax import lax
from jax.experimental import pallas as pl
from jax.experimental.pallas import tpu as pltpu
from jax.experimental.pallas import tpu_sc as plsc

N_FIELDS = 26
VOCAB = 100000
EMB = 16
OHE = 100
HID = 128
BATCH = 16384
EMB_FEAT = N_FIELDS * EMB
TOTAL_ROWS = BATCH * N_FIELDS

_NC = 2
_NS = 16
_NW = _NC * _NS

_C = 1408
_CHUNKS = 99968 // _C
_TAIL0 = _CHUNKS * _C
_TAILW = VOCAB - _TAIL0
_N_U1 = N_FIELDS * _CHUNKS
_U1_PER_W = (_N_U1 + _NW - 1) // _NW


_U1_GROUP = 6
_U1_ROUNDS = (_U1_PER_W + _U1_GROUP - 1) // _U1_GROUP


def _relayout_body(
    tab_hbm, out_hbm, in0, in1, in2, ot0, ot1, in_t, out_t, si0, si1, si2, so0, so1
):
    wid = lax.axis_index("s") * _NC + lax.axis_index("c")
    rows_iota = lax.broadcasted_iota(jnp.int32, (EMB,), 0)
    ins = ((in0, si0), (in1, si1), (in2, si2))
    outs = ((ot0, so0), (ot1, so1))

    def in_slice(u):
        f = u // _CHUNKS
        c0 = (u % _CHUNKS) * _C
        return tab_hbm.at[pl.ds(f * EMB, EMB), pl.ds(c0, _C)]

    def out_slice(u):
        f = u // _CHUNKS
        c0 = (u % _CHUNKS) * _C
        return out_hbm.at[pl.ds((f * VOCAB + c0) * EMB, _C * EMB)]

    def transpose(src, dst, width):
        @plsc.parallel_loop(0, width // EMB, unroll=4, carry=rows_iota * EMB)
        def body(cb, idx0):
            for j in range(EMB):
                vec = src[j, pl.ds(cb * EMB, EMB)]
                plsc.store_scatter(dst, [idx0 + j], vec)
            return idx0 + EMB * EMB

    for b in range(3):
        u = wid + _NW * b

        @pl.when(u < _N_U1)
        def _():
            pltpu.async_copy(in_slice(u), ins[b][0], ins[b][1])

    def round_body(m, carry):
        for b in range(_U1_GROUP):
            k = _U1_GROUP * m + b
            u = wid + _NW * k
            u_prev = u - 2 * _NW
            u_next = u + 3 * _NW
            ib, ob = ins[b % 3], outs[b % 2]

            @pl.when((k >= 2) & (u_prev < _N_U1))
            def _():
                pltpu.make_async_copy(ob[0], out_slice(u_prev), ob[1]).wait()

            @pl.when(u < _N_U1)
            def _():
                pltpu.make_async_copy(in_slice(u), ib[0], ib[1]).wait()
                transpose(ib[0], ob[0], _C)
                pltpu.async_copy(ob[0], out_slice(u), ob[1])

            @pl.when(u_next < _N_U1)
            def _():
                pltpu.async_copy(in_slice(u_next), ib[0], ib[1])

        return carry

    lax.fori_loop(0, _U1_ROUNDS, round_body, 0)

    @pl.when(wid < N_FIELDS)
    def _():
        f = wid
        pltpu.sync_copy(tab_hbm.at[pl.ds(f * EMB, EMB), pl.ds(_TAIL0, _TAILW)], in_t)
        transpose(in_t, out_t, _TAILW)
        pltpu.sync_copy(out_t, out_hbm.at[pl.ds((f * VOCAB + _TAIL0) * EMB, _TAILW * EMB)])


_relayout = functools.partial(
    pl.kernel,
    mesh=plsc.VectorSubcoreMesh(core_axis_name="c", subcore_axis_name="s"),
    out_type=jax.ShapeDtypeStruct((N_FIELDS * VOCAB * EMB,), jnp.float32),
    scratch_types=[
        pltpu.VMEM((EMB, _C), jnp.float32),
        pltpu.VMEM((EMB, _C), jnp.float32),
        pltpu.VMEM((EMB, _C), jnp.float32),
        pltpu.VMEM((_C * EMB,), jnp.float32),
        pltpu.VMEM((_C * EMB,), jnp.float32),
        pltpu.VMEM((EMB, _TAILW), jnp.float32),
        pltpu.VMEM((_TAILW * EMB,), jnp.float32),
        pltpu.SemaphoreType.DMA,
        pltpu.SemaphoreType.DMA,
        pltpu.SemaphoreType.DMA,
        pltpu.SemaphoreType.DMA,
        pltpu.SemaphoreType.DMA,
    ],
    compiler_params=pltpu.CompilerParams(
        use_tc_tiling_on_sc=True, needs_layout_passes=False
    ),
)(_relayout_body)


_PER_W = TOTAL_ROWS // _NW
_CHUNK = 3328
_N_CHUNKS = _PER_W // _CHUNK


_G_CHUNK = 1664
_G_N = _PER_W // _G_CHUNK


def _gather_body(table_hbm, idx_hbm, out_hbm, i0, i1, r0, r1, gi0, gi1, gr0, gr1):
    wid = lax.axis_index("s") * _NC + lax.axis_index("c")
    base = wid * _PER_W
    idx = ((i0, gi0), (i1, gi1))
    rows = ((r0, gr0), (r1, gr1))

    def islice(c):
        return idx_hbm.at[pl.ds(base + c * _G_CHUNK, _G_CHUNK)]

    def oslice(c):
        return out_hbm.at[pl.ds(base + c * _G_CHUNK, _G_CHUNK)]

    for b in range(2):
        pltpu.async_copy(islice(b), idx[b][0], idx[b][1])
    for c in range(_G_N):
        b = c % 2
        if c >= 2:
            pltpu.make_async_copy(rows[b][0], oslice(c - 2), rows[b][1]).wait()
        pltpu.make_async_copy(islice(c), idx[b][0], idx[b][1]).wait()
        pltpu.async_copy(table_hbm.at[idx[b][0]], rows[b][0], rows[b][1]).wait()
        pltpu.async_copy(rows[b][0], oslice(c), rows[b][1])
        if c + 2 < _G_N:
            pltpu.async_copy(islice(c + 2), idx[b][0], idx[b][1])
    for c in (_G_N - 2, _G_N - 1):
        pltpu.make_async_copy(rows[c % 2][0], oslice(c), rows[c % 2][1]).wait()


_gather = functools.partial(
    pl.kernel,
    mesh=plsc.VectorSubcoreMesh(core_axis_name="c", subcore_axis_name="s"),
    out_type=jax.ShapeDtypeStruct((TOTAL_ROWS, EMB), jnp.float32),
    scratch_types=[
        pltpu.VMEM((_G_CHUNK,), jnp.int32),
        pltpu.VMEM((_G_CHUNK,), jnp.int32),
        pltpu.VMEM((_G_CHUNK, EMB), jnp.float32),
        pltpu.VMEM((_G_CHUNK, EMB), jnp.float32),
        pltpu.SemaphoreType.DMA,
        pltpu.SemaphoreType.DMA,
        pltpu.SemaphoreType.DMA,
        pltpu.SemaphoreType.DMA,
    ],
    compiler_params=pltpu.CompilerParams(use_tc_tiling_on_sc=False),
)(_gather_body)


_BM = 2048


def _mm0_body(o_ref, w2_ref, b_ref, out_ref):
    out_ref[...] = (
        jnp.dot(o_ref[...], w2_ref[...], preferred_element_type=jnp.float32)
        + b_ref[...]
    )


_mm0 = pl.pallas_call(
    _mm0_body,
    grid=(BATCH // _BM,),
    in_specs=[
        pl.BlockSpec((_BM, OHE), lambda i: (i, 0)),
        pl.BlockSpec((OHE, HID), lambda i: (0, 0)),
        pl.BlockSpec((1, HID), lambda i: (0, 0)),
    ],
    out_specs=pl.BlockSpec((_BM, HID), lambda i: (i, 0)),
    out_shape=jax.ShapeDtypeStruct((BATCH, HID), jnp.float32),
)


def _mm_body(e_ref, h0_ref, w1_ref, out_ref):
    acc = jnp.dot(e_ref[...], w1_ref[...], preferred_element_type=jnp.float32)
    out_ref[...] = acc + h0_ref[...]


_mm = pl.pallas_call(
    _mm_body,
    grid=(BATCH // _BM,),
    in_specs=[
        pl.BlockSpec((_BM, EMB_FEAT), lambda i: (i, 0)),
        pl.BlockSpec((_BM, HID), lambda i: (i, 0)),
        pl.BlockSpec((EMB_FEAT, HID), lambda i: (0, 0)),
    ],
    out_specs=pl.BlockSpec((_BM, HID), lambda i: (i, 0)),
    out_shape=jax.ShapeDtypeStruct((BATCH, HID), jnp.float32),
)


@jax.jit
def kernel(embed_idx, ohes, tables, W, b):
    tab_t = tables.transpose(0, 2, 1).reshape(EMB_FEAT, VOCAB)
    table_lin = _relayout(tab_t).reshape(N_FIELDS * VOCAB, EMB)
    offs = jnp.arange(N_FIELDS, dtype=jnp.int32) * VOCAB
    flat_idx = (embed_idx.astype(jnp.int32) + offs[None, :]).reshape(TOTAL_ROWS)
    h0 = _mm0(ohes, W[EMB_FEAT:], b.reshape(1, HID))
    e = _gather(table_lin, flat_idx)
    e = e.reshape(BATCH, EMB_FEAT)
    return _mm(e, h0, W[:EMB_FEAT])
